# Initial kernel scaffold; baseline (speedup 1.0000x reference)
import jax
import jax.numpy as jnp
from jax.experimental import pallas as pl


def kernel(x, embeddings):
    raise NotImplementedError("write your pallas kernel here")


# speedup vs baseline: 1.1963x; 1.1963x over previous
import functools

import jax
import jax.numpy as jnp
from jax import lax
from jax.experimental import pallas as pl
from jax.experimental.pallas import tpu as pltpu
from jax.experimental.pallas import tpu_sc as plsc

_NE = 1024
_D = 64
_TB = 1024
_NC = 2
_NS = 16
_NW = _NC * _NS
_LOSS_SCALE = 1.25


def _dist_argmin_body(x_ref, e_ref, idx_ref, msum_ref):
    xb = x_ref[...]
    e = e_ref[...]
    sim = lax.dot_general(xb, e, (((1,), (1,)), ((), ())),
                          preferred_element_type=jnp.float32)
    xn = jnp.sum(xb * xb, axis=1, keepdims=True)
    en = jnp.sum(e * e, axis=1)
    dist = xn + en[None, :] - 2.0 * sim
    idx_ref[0, 0, :] = jnp.argmin(dist, axis=1).astype(jnp.int32)
    msum_ref[...] = jnp.sum(jnp.min(dist, axis=1)).reshape(1, 1, 1)


def _dist_argmin(flat, embeddings):
    n = flat.shape[0]
    nblk = n // _TB
    idx3, msum = pl.pallas_call(
        _dist_argmin_body,
        grid=(nblk,),
        in_specs=[
            pl.BlockSpec((_TB, _D), lambda i: (i, 0)),
            pl.BlockSpec((_NE, _D), lambda i: (0, 0)),
        ],
        out_specs=[
            pl.BlockSpec((1, 1, _TB), lambda i: (i, 0, 0)),
            pl.BlockSpec((1, 1, 1), lambda i: (i, 0, 0)),
        ],
        out_shape=[
            jax.ShapeDtypeStruct((nblk, 1, _TB), jnp.int32),
            jax.ShapeDtypeStruct((nblk, 1, 1), jnp.float32),
        ],
    )(flat, embeddings)
    return idx3.reshape(-1), jnp.sum(msum)


def _make_sc_gather(n_tokens):
    bpw = n_tokens // _NW

    @functools.partial(
        pl.kernel,
        mesh=plsc.VectorSubcoreMesh(core_axis_name="c", subcore_axis_name="s"),
        out_type=jax.ShapeDtypeStruct((n_tokens, _D), jnp.float32),
        scratch_types=[
            pltpu.VMEM((bpw,), jnp.int32),
            pltpu.VMEM((bpw, _D), jnp.float32),
            pltpu.SemaphoreType.DMA,
        ],
        compiler_params=pltpu.CompilerParams(use_tc_tiling_on_sc=False),
    )
    def sc_gather(table_hbm, idx_hbm, out_hbm, idx_v, rows_v, sem):
        wid = lax.axis_index("s") * _NC + lax.axis_index("c")
        base = wid * bpw
        pltpu.sync_copy(idx_hbm.at[pl.ds(base, bpw)], idx_v)
        pltpu.async_copy(table_hbm.at[idx_v], rows_v, sem).wait()
        pltpu.sync_copy(rows_v, out_hbm.at[pl.ds(base, bpw)])

    return sc_gather


def kernel(x, embeddings):
    input_shape = x.shape
    flat = x.reshape(-1, _D)
    n = flat.shape[0]
    idx, min_sum = _dist_argmin(flat, embeddings)
    quant_flat = _make_sc_gather(n)(embeddings, idx)
    out = quant_flat.reshape(input_shape)
    vq_loss = _LOSS_SCALE * min_sum / (n * _D)
    return out, vq_loss
